# baseline (speedup 1.0000x reference)
import jax
import jax.numpy as jnp
from jax import lax
from jax.experimental import pallas as pl
from jax.experimental.pallas import tpu as pltpu
from jax.experimental.pallas import tpu_sc as plsc

N_CORES = 2
N_SUBCORES = 16
N_WORKERS = N_CORES * N_SUBCORES
BATCH = 16384
DIM = 32
NROWS = 1000000
ROWS_PER_W = BATCH // N_WORKERS
CHUNK = 128
N_CHUNKS = ROWS_PER_W // CHUNK
GROUPS = CHUNK // 16

BLK_U = 512
DET_GRID = (NROWS + BLK_U - 1) // BLK_U
SROWS = ((NROWS + BLK_U - 1) // BLK_U) * (BLK_U // 4)


def _detile_body(ut_ref, it_ref, uo_ref, io_ref):
    eye = jnp.eye(DIM, dtype=jnp.float32)
    for ref, o in ((ut_ref, uo_ref), (it_ref, io_ref)):
        x = ref[...]
        o[...] = jnp.concatenate(
            [lax.dot_general(x[:, a * 128:(a + 1) * 128], eye,
                             (((0,), (0,)), ((), ())),
                             preferred_element_type=jnp.float32)
             for a in range(4)],
            axis=1)


def _detile(ut_t, it_t):
    return pl.pallas_call(
        _detile_body,
        grid=(DET_GRID,),
        in_specs=[pl.BlockSpec((DIM, BLK_U), lambda c: (0, c)),
                  pl.BlockSpec((DIM, BLK_U), lambda c: (0, c))],
        out_specs=[pl.BlockSpec((BLK_U // 4, 128), lambda c: (c, 0)),
                   pl.BlockSpec((BLK_U // 4, 128), lambda c: (c, 0))],
        out_shape=[jax.ShapeDtypeStruct((SROWS, 128), jnp.float32),
                   jax.ShapeDtypeStruct((SROWS, 128), jnp.float32)],
    )(ut_t, it_t)


def _gmf_body(user_h, item_h, ut_h, it_h, w_h, b_h, out_h,
              uidx, iidx, urow, irow, ucol, icol,
              ubuf, vbuf, wv, bv, outv, sem):
    wid = lax.axis_index("s") * N_CORES + lax.axis_index("c")
    base = wid * ROWS_PER_W

    pltpu.sync_copy(user_h.at[pl.ds(base, ROWS_PER_W)], uidx)
    pltpu.sync_copy(item_h.at[pl.ds(base, ROWS_PER_W)], iidx)
    pltpu.sync_copy(w_h, wv)
    pltpu.sync_copy(b_h, bv)

    for k in range(ROWS_PER_W // 16):
        sl = pl.ds(k * 16, 16)
        u = uidx[sl]
        i = iidx[sl]
        urow[sl] = lax.shift_left(lax.shift_right_logical(u, 9), 7) \
            + jnp.bitwise_and(u, 127)
        irow[sl] = lax.shift_left(lax.shift_right_logical(i, 9), 7) \
            + jnp.bitwise_and(i, 127)
        ucol[sl] = lax.shift_left(
            jnp.bitwise_and(lax.shift_right_logical(u, 7), 3), 5)
        icol[sl] = lax.shift_left(
            jnp.bitwise_and(lax.shift_right_logical(i, 7), 3), 5)

    wlo = wv[pl.ds(0, 16)]
    whi = wv[pl.ds(16, 16)]
    w_s = [wlo[d] for d in range(16)] + [whi[d] for d in range(16)]
    bvec = bv[...]
    iota16 = lax.iota(jnp.int32, 16)

    for j in range(N_CHUNKS):
        csl = pl.ds(j * CHUNK, CHUNK)
        du = pltpu.async_copy(ut_h.at[urow.at[csl]], ubuf, sem)
        dv = pltpu.async_copy(it_h.at[irow.at[csl]], vbuf, sem)
        du.wait()
        dv.wait()

        def group(g, carry):
            rows = g * 16 + iota16
            ucols = ucol[pl.ds(j * CHUNK + g * 16, 16)]
            icols = icol[pl.ds(j * CHUNK + g * 16, 16)]
            acc = bvec
            for d in range(DIM):
                gu = plsc.load_gather(ubuf, [rows, ucols + d])
                gv = plsc.load_gather(vbuf, [rows, icols + d])
                acc = acc + gu * gv * w_s[d]
            y = 1.0 / (1.0 + jnp.exp(-acc))
            outv[pl.ds(j * CHUNK + g * 16, 16)] = y
            return carry

        lax.fori_loop(0, GROUPS, group, 0)

    pltpu.sync_copy(outv, out_h.at[pl.ds(base, ROWS_PER_W)])


def _gmf(user, item, ut4, it4, w32, b16):
    mesh = plsc.VectorSubcoreMesh(core_axis_name="c", subcore_axis_name="s",
                                  num_cores=N_CORES, num_subcores=N_SUBCORES)
    run = pl.kernel(
        _gmf_body,
        out_type=jax.ShapeDtypeStruct((BATCH,), jnp.float32),
        mesh=mesh,
        scratch_types=[
            pltpu.VMEM((ROWS_PER_W,), jnp.int32),
            pltpu.VMEM((ROWS_PER_W,), jnp.int32),
            pltpu.VMEM((ROWS_PER_W,), jnp.int32),
            pltpu.VMEM((ROWS_PER_W,), jnp.int32),
            pltpu.VMEM((ROWS_PER_W,), jnp.int32),
            pltpu.VMEM((ROWS_PER_W,), jnp.int32),
            pltpu.VMEM((CHUNK, 128), jnp.float32),
            pltpu.VMEM((CHUNK, 128), jnp.float32),
            pltpu.VMEM((DIM,), jnp.float32),
            pltpu.VMEM((16,), jnp.float32),
            pltpu.VMEM((ROWS_PER_W,), jnp.float32),
            pltpu.SemaphoreType.DMA,
        ],
        compiler_params=pltpu.CompilerParams(needs_layout_passes=False,
                                             use_tc_tiling_on_sc=True),
    )
    return run(user, item, ut4, it4, w32, b16)


@jax.jit
def _run(user, item, ut_t, it_t, w32, b16):
    ut4, it4 = _detile(ut_t, it_t)
    return _gmf(user, item, ut4, it4, w32, b16)


def kernel(user, item, users_table, items_table, W, b):
    return _run(user, item, users_table.T, items_table.T,
                W.reshape(DIM), jnp.broadcast_to(b.reshape(()), (16,)))


# speedup vs baseline: 2.6331x; 2.6331x over previous
import jax
import jax.numpy as jnp
from jax import lax
from jax.experimental import pallas as pl
from jax.experimental.pallas import tpu as pltpu
from jax.experimental.pallas import tpu_sc as plsc

N_CORES = 2
N_SUBCORES = 16
N_WORKERS = N_CORES * N_SUBCORES
BATCH = 16384
DIM = 32
NROWS = 1000000
ROWS_PER_W = BATCH // N_WORKERS
CHUNK = 128
N_CHUNKS = ROWS_PER_W // CHUNK
GROUPS = CHUNK // 16

BLK_U = 4096
DET_GRID = (NROWS + BLK_U - 1) // BLK_U
SROWS = ((NROWS + BLK_U - 1) // BLK_U) * (BLK_U // 4)


def _detile_body(ut_ref, it_ref, uo_ref, io_ref):
    q = BLK_U // 4
    for ref, o in ((ut_ref, uo_ref), (it_ref, io_ref)):
        x = ref[...]
        o[...] = jnp.concatenate(
            [jnp.transpose(x[:, a * q:(a + 1) * q]) for a in range(4)],
            axis=1)


def _detile(ut_t, it_t):
    return pl.pallas_call(
        _detile_body,
        grid=(DET_GRID,),
        in_specs=[pl.BlockSpec((DIM, BLK_U), lambda c: (0, c)),
                  pl.BlockSpec((DIM, BLK_U), lambda c: (0, c))],
        out_specs=[pl.BlockSpec((BLK_U // 4, 128), lambda c: (c, 0)),
                   pl.BlockSpec((BLK_U // 4, 128), lambda c: (c, 0))],
        out_shape=[jax.ShapeDtypeStruct((SROWS, 128), jnp.float32),
                   jax.ShapeDtypeStruct((SROWS, 128), jnp.float32)],
    )(ut_t, it_t)


def _gmf_body(user_h, item_h, ut_h, it_h, w_h, b_h, out_h,
              uidx, iidx, urow, irow, ucol, icol,
              ubuf, vbuf, wv, bv, outv, sem):
    wid = lax.axis_index("s") * N_CORES + lax.axis_index("c")
    base = wid * ROWS_PER_W

    pltpu.sync_copy(user_h.at[pl.ds(base, ROWS_PER_W)], uidx)
    pltpu.sync_copy(item_h.at[pl.ds(base, ROWS_PER_W)], iidx)
    pltpu.sync_copy(w_h, wv)
    pltpu.sync_copy(b_h, bv)

    for k in range(ROWS_PER_W // 16):
        sl = pl.ds(k * 16, 16)
        u = uidx[sl]
        i = iidx[sl]
        urow[sl] = lax.shift_left(lax.shift_right_logical(u, 12), 10) \
            + jnp.bitwise_and(u, 1023)
        irow[sl] = lax.shift_left(lax.shift_right_logical(i, 12), 10) \
            + jnp.bitwise_and(i, 1023)
        ucol[sl] = lax.shift_left(
            jnp.bitwise_and(lax.shift_right_logical(u, 10), 3), 5)
        icol[sl] = lax.shift_left(
            jnp.bitwise_and(lax.shift_right_logical(i, 10), 3), 5)

    wlo = wv[pl.ds(0, 16)]
    whi = wv[pl.ds(16, 16)]
    w_s = [wlo[d] for d in range(16)] + [whi[d] for d in range(16)]
    bvec = bv[...]
    iota16 = lax.iota(jnp.int32, 16)

    for j in range(N_CHUNKS):
        csl = pl.ds(j * CHUNK, CHUNK)
        du = pltpu.async_copy(ut_h.at[urow.at[csl]], ubuf, sem)
        dv = pltpu.async_copy(it_h.at[irow.at[csl]], vbuf, sem)
        du.wait()
        dv.wait()

        def group(g, carry):
            rows = g * 16 + iota16
            ucols = ucol[pl.ds(j * CHUNK + g * 16, 16)]
            icols = icol[pl.ds(j * CHUNK + g * 16, 16)]
            acc = bvec
            for d in range(DIM):
                gu = plsc.load_gather(ubuf, [rows, ucols + d])
                gv = plsc.load_gather(vbuf, [rows, icols + d])
                acc = acc + gu * gv * w_s[d]
            y = 1.0 / (1.0 + jnp.exp(-acc))
            outv[pl.ds(j * CHUNK + g * 16, 16)] = y
            return carry

        lax.fori_loop(0, GROUPS, group, 0)

    pltpu.sync_copy(outv, out_h.at[pl.ds(base, ROWS_PER_W)])


def _gmf(user, item, ut4, it4, w32, b16):
    mesh = plsc.VectorSubcoreMesh(core_axis_name="c", subcore_axis_name="s",
                                  num_cores=N_CORES, num_subcores=N_SUBCORES)
    run = pl.kernel(
        _gmf_body,
        out_type=jax.ShapeDtypeStruct((BATCH,), jnp.float32),
        mesh=mesh,
        scratch_types=[
            pltpu.VMEM((ROWS_PER_W,), jnp.int32),
            pltpu.VMEM((ROWS_PER_W,), jnp.int32),
            pltpu.VMEM((ROWS_PER_W,), jnp.int32),
            pltpu.VMEM((ROWS_PER_W,), jnp.int32),
            pltpu.VMEM((ROWS_PER_W,), jnp.int32),
            pltpu.VMEM((ROWS_PER_W,), jnp.int32),
            pltpu.VMEM((CHUNK, 128), jnp.float32),
            pltpu.VMEM((CHUNK, 128), jnp.float32),
            pltpu.VMEM((DIM,), jnp.float32),
            pltpu.VMEM((16,), jnp.float32),
            pltpu.VMEM((ROWS_PER_W,), jnp.float32),
            pltpu.SemaphoreType.DMA,
        ],
        compiler_params=pltpu.CompilerParams(needs_layout_passes=False,
                                             use_tc_tiling_on_sc=True),
    )
    return run(user, item, ut4, it4, w32, b16)


@jax.jit
def _run(user, item, ut_t, it_t, w32, b16):
    ut4, it4 = _detile(ut_t, it_t)
    return _gmf(user, item, ut4, it4, w32, b16)


def kernel(user, item, users_table, items_table, W, b):
    return _run(user, item, users_table.T, items_table.T,
                W.reshape(DIM), jnp.broadcast_to(b.reshape(()), (16,)))


# speedup vs baseline: 2.7039x; 1.0269x over previous
import jax
import jax.numpy as jnp
from jax import lax
from jax.experimental import pallas as pl
from jax.experimental.pallas import tpu as pltpu
from jax.experimental.pallas import tpu_sc as plsc

N_CORES = 2
N_SUBCORES = 16
N_WORKERS = N_CORES * N_SUBCORES
BATCH = 16384
DIM = 32
NROWS = 1000000
ROWS_PER_W = BATCH // N_WORKERS
CHUNK = 128
N_CHUNKS = ROWS_PER_W // CHUNK
GROUPS = CHUNK // 16

BLK_U = 16384
DET_GRID = (NROWS + BLK_U - 1) // BLK_U
SROWS = ((NROWS + BLK_U - 1) // BLK_U) * (BLK_U // 4)


def _detile_body(ut_ref, it_ref, uo_ref, io_ref):
    q = BLK_U // 4
    for ref, o in ((ut_ref, uo_ref), (it_ref, io_ref)):
        x = ref[...]
        o[...] = jnp.concatenate(
            [jnp.transpose(x[:, a * q:(a + 1) * q]) for a in range(4)],
            axis=1)


def _detile(ut_t, it_t):
    return pl.pallas_call(
        _detile_body,
        grid=(DET_GRID,),
        in_specs=[pl.BlockSpec((DIM, BLK_U), lambda c: (0, c)),
                  pl.BlockSpec((DIM, BLK_U), lambda c: (0, c))],
        out_specs=[pl.BlockSpec((BLK_U // 4, 128), lambda c: (c, 0)),
                   pl.BlockSpec((BLK_U // 4, 128), lambda c: (c, 0))],
        out_shape=[jax.ShapeDtypeStruct((SROWS, 128), jnp.float32),
                   jax.ShapeDtypeStruct((SROWS, 128), jnp.float32)],
    )(ut_t, it_t)


def _gmf_body(user_h, item_h, ut_h, it_h, w_h, b_h, out_h,
              uidx, iidx, urow, irow, ucol, icol,
              ubuf, vbuf, wv, bv, outv, sem):
    wid = lax.axis_index("s") * N_CORES + lax.axis_index("c")
    base = wid * ROWS_PER_W

    pltpu.sync_copy(user_h.at[pl.ds(base, ROWS_PER_W)], uidx)
    pltpu.sync_copy(item_h.at[pl.ds(base, ROWS_PER_W)], iidx)
    pltpu.sync_copy(w_h, wv)
    pltpu.sync_copy(b_h, bv)

    for k in range(ROWS_PER_W // 16):
        sl = pl.ds(k * 16, 16)
        u = uidx[sl]
        i = iidx[sl]
        urow[sl] = lax.shift_left(lax.shift_right_logical(u, 14), 12) \
            + jnp.bitwise_and(u, 4095)
        irow[sl] = lax.shift_left(lax.shift_right_logical(i, 14), 12) \
            + jnp.bitwise_and(i, 4095)
        ucol[sl] = lax.shift_left(
            jnp.bitwise_and(lax.shift_right_logical(u, 12), 3), 5)
        icol[sl] = lax.shift_left(
            jnp.bitwise_and(lax.shift_right_logical(i, 12), 3), 5)

    wlo = wv[pl.ds(0, 16)]
    whi = wv[pl.ds(16, 16)]
    w_s = [wlo[d] for d in range(16)] + [whi[d] for d in range(16)]
    bvec = bv[...]
    iota16 = lax.iota(jnp.int32, 16)

    for j in range(N_CHUNKS):
        csl = pl.ds(j * CHUNK, CHUNK)
        du = pltpu.async_copy(ut_h.at[urow.at[csl]], ubuf, sem)
        dv = pltpu.async_copy(it_h.at[irow.at[csl]], vbuf, sem)
        du.wait()
        dv.wait()

        def group(g, carry):
            rows = g * 16 + iota16
            ucols = ucol[pl.ds(j * CHUNK + g * 16, 16)]
            icols = icol[pl.ds(j * CHUNK + g * 16, 16)]
            acc = bvec
            for d in range(DIM):
                gu = plsc.load_gather(ubuf, [rows, ucols + d])
                gv = plsc.load_gather(vbuf, [rows, icols + d])
                acc = acc + gu * gv * w_s[d]
            y = 1.0 / (1.0 + jnp.exp(-acc))
            outv[pl.ds(j * CHUNK + g * 16, 16)] = y
            return carry

        lax.fori_loop(0, GROUPS, group, 0)

    pltpu.sync_copy(outv, out_h.at[pl.ds(base, ROWS_PER_W)])


def _gmf(user, item, ut4, it4, w32, b16):
    mesh = plsc.VectorSubcoreMesh(core_axis_name="c", subcore_axis_name="s",
                                  num_cores=N_CORES, num_subcores=N_SUBCORES)
    run = pl.kernel(
        _gmf_body,
        out_type=jax.ShapeDtypeStruct((BATCH,), jnp.float32),
        mesh=mesh,
        scratch_types=[
            pltpu.VMEM((ROWS_PER_W,), jnp.int32),
            pltpu.VMEM((ROWS_PER_W,), jnp.int32),
            pltpu.VMEM((ROWS_PER_W,), jnp.int32),
            pltpu.VMEM((ROWS_PER_W,), jnp.int32),
            pltpu.VMEM((ROWS_PER_W,), jnp.int32),
            pltpu.VMEM((ROWS_PER_W,), jnp.int32),
            pltpu.VMEM((CHUNK, 128), jnp.float32),
            pltpu.VMEM((CHUNK, 128), jnp.float32),
            pltpu.VMEM((DIM,), jnp.float32),
            pltpu.VMEM((16,), jnp.float32),
            pltpu.VMEM((ROWS_PER_W,), jnp.float32),
            pltpu.SemaphoreType.DMA,
        ],
        compiler_params=pltpu.CompilerParams(needs_layout_passes=False,
                                             use_tc_tiling_on_sc=True),
    )
    return run(user, item, ut4, it4, w32, b16)


@jax.jit
def _run(user, item, ut_t, it_t, w32, b16):
    ut4, it4 = _detile(ut_t, it_t)
    return _gmf(user, item, ut4, it4, w32, b16)


def kernel(user, item, users_table, items_table, W, b):
    return _run(user, item, users_table.T, items_table.T,
                W.reshape(DIM), jnp.broadcast_to(b.reshape(()), (16,)))
